# baseline (speedup 1.0000x reference)
import jax
import jax.numpy as jnp
from jax import lax
from jax.experimental import pallas as pl
from jax.experimental.pallas import tpu as pltpu
from jax.experimental.pallas import tpu_sc as plsc

H = 128
W = 128
SY = 4
SX = 4
N = H * W
C = 256
ND = (H // SY) * (W // SX)
NDP = 1032
NP8 = N + 8
R_MERGE = min(N // 2, N - ND)
BN = 1024
NB = N // BN
NSC = 2
NTILE = 16
TOK_TILE = N // NTILE


def _scores_body(xb_ref, xd_ref, nmax_ref, nidx_ref):
    xb = xb_ref[0]
    xd = xd_ref[0]
    mb = xb / (jnp.sqrt(jnp.sum(xb * xb, axis=1, keepdims=True)) + 1e-6)
    md = xd / (jnp.sqrt(jnp.sum(xd * xd, axis=1, keepdims=True)) + 1e-6)
    s = lax.dot_general(mb, md, (((1,), (1,)), ((), ())))
    nmax = jnp.max(s, axis=1)
    iota = lax.broadcasted_iota(jnp.int32, (BN, ND), 1)
    nidx = jnp.min(jnp.where(s == nmax[:, None], iota, ND), axis=1)
    nmax_ref[0, 0] = nmax
    nidx_ref[0, 0] = nidx.astype(jnp.int32)


def _select_body(nm_ref, ni_ref, sc_ref, gi_ref, sp_ref):
    b = pl.program_id(0)
    nm = nm_ref[0]
    ni = ni_ref[0]
    ri = lax.broadcasted_iota(jnp.int32, (H, W), 0)
    ci = lax.broadcasted_iota(jnp.int32, (H, W), 1)
    is_dst = ((ri % SY) == 0) & ((ci % SX) == 0)
    bits = lax.bitcast_convert_type(nm, jnp.int32)
    v = bits ^ (lax.shift_right_arithmetic(bits, 31) & jnp.int32(0x7FFFFFFF))
    min32 = jnp.int32(-2147483648)
    v = jnp.where(is_dst, min32, v)
    r = jnp.int32(R_MERGE)
    cnt_pos = jnp.sum((v >= 0).astype(jnp.int32))
    bucket_pos = cnt_pos >= r
    in_bucket = ((v >= 0) == bucket_pos) & jnp.logical_not(is_dst)
    key = jnp.where(in_bucket, v & jnp.int32(0x7FFFFFFF), jnp.int32(-1))
    rr = jnp.where(bucket_pos, r, r - cnt_pos)

    def body(k, prefix):
        cand = prefix | lax.shift_left(jnp.int32(1), jnp.int32(30) - k)
        cnt = jnp.sum((key >= cand).astype(jnp.int32))
        return jnp.where(cnt >= rr, cand, prefix)

    t = lax.fori_loop(0, 31, body, jnp.int32(0))
    tv = jnp.where(bucket_pos, t, t | min32)
    not_dst = jnp.logical_not(is_dst)
    gt = (v > tv) & not_dst
    eq = (v == tv) & not_dst
    cnt_gt = jnp.sum(gt.astype(jnp.int32))
    need = (r - cnt_gt).astype(jnp.float32)
    e = eq.astype(jnp.float32)
    tri_incl = (ri <= ci).astype(jnp.float32)
    incl = lax.dot_general(e, tri_incl, (((1,), (0,)), ((), ())),
                           precision=lax.Precision.HIGHEST)
    row_tot = incl[:, W - 1:W]
    tri_strict = (ri > ci).astype(jnp.float32)
    off = lax.dot_general(tri_strict, row_tot, (((1,), (0,)), ((), ())),
                          precision=lax.Precision.HIGHEST)
    rank_excl = incl - e + off
    merged = gt | (eq & (rank_excl < need))
    sc_ref[0] = jnp.where(merged, ni, jnp.int32(ND))
    dbin = (ri // SY) * (W // SX) + (ci // SX)
    g = jnp.where(is_dst, dbin, jnp.where(merged, ni, jnp.int32(-1)))
    sel = g >= 0
    gi_ref[0] = b * NDP + jnp.where(sel, g, jnp.int32(ND))
    p = ri * W + ci
    sp_ref[0] = b * NP8 + jnp.where(sel, p, jnp.int32(N))


def _scatter_body(xb_ref, si_ref, xdp_ref, dm_ref, acc_ref, cnt_ref):
    nb = pl.program_id(1)

    @pl.when(nb == 0)
    def _():
        acc_ref[...] = jnp.zeros_like(acc_ref)
        cnt_ref[...] = jnp.zeros_like(cnt_ref)

    xb = xb_ref[0]
    si = jnp.broadcast_to(si_ref[0], (ND, BN))
    iota = lax.broadcasted_iota(jnp.int32, (ND, BN), 0)
    oht = (si == iota).astype(jnp.float32)
    acc_ref[pl.ds(0, ND), :] += lax.dot_general(
        oht, xb, (((1,), (0,)), ((), ())), precision=lax.Precision.HIGHEST)
    cnt_ref[pl.ds(0, ND), :] += lax.dot_general(
        oht, jnp.ones((BN, 128), jnp.float32), (((1,), (0,)), ((), ())),
        precision=lax.Precision.HIGHEST)

    @pl.when(nb == NB - 1)
    def _():
        dm_ref[0] = (xdp_ref[0] + acc_ref[...]) / (1.0 + cnt_ref[:, 0:1])


def _sc_unmerge_body(x_hbm, gi_hbm, sp_hbm, dmf_hbm, out_hbm,
                     xbuf, gbuf, gixbuf, spxbuf, sem):
    c = lax.axis_index("c")
    s = lax.axis_index("s")
    pltpu.sync_copy(gi_hbm.at[c, pl.ds(s * 8, 8)], gixbuf)
    pltpu.sync_copy(sp_hbm.at[c, pl.ds(s * 8, 8)], spxbuf)
    base = s * TOK_TILE
    for k in range(8):
        pltpu.sync_copy(x_hbm.at[c, pl.ds(base + k * 128, 128)], xbuf)
        pltpu.sync_copy(xbuf, out_hbm.at[pl.ds(c * NP8 + base + k * 128, 128)])
        pltpu.async_copy(dmf_hbm.at[gixbuf.at[k]], gbuf, sem).wait()
        pltpu.sync_copy(gbuf, out_hbm.at[spxbuf.at[k]])


_sc_unmerge = pl.kernel(
    _sc_unmerge_body,
    out_type=jax.ShapeDtypeStruct((NSC * NP8, C), jnp.float32),
    mesh=plsc.VectorSubcoreMesh(core_axis_name="c", subcore_axis_name="s"),
    scratch_types=[
        pltpu.VMEM((128, C), jnp.float32),
        pltpu.VMEM((128, C), jnp.float32),
        pltpu.VMEM((8, 128), jnp.int32),
        pltpu.VMEM((8, 128), jnp.int32),
        pltpu.SemaphoreType.DMA,
    ],
)


def kernel(x):
    B = x.shape[0]
    x_dst = x.reshape(B, H // SY, SY, W // SX, SX, C)[:, :, 0, :, 0, :]
    x_dst = x_dst.reshape(B, ND, C)

    nmax, nidx = pl.pallas_call(
        _scores_body,
        grid=(B, NB),
        in_specs=[
            pl.BlockSpec((1, BN, C), lambda b, nb: (b, nb, 0)),
            pl.BlockSpec((1, ND, C), lambda b, nb: (b, 0, 0)),
        ],
        out_specs=[
            pl.BlockSpec((1, 1, BN), lambda b, nb: (b * NB + nb, 0, 0)),
            pl.BlockSpec((1, 1, BN), lambda b, nb: (b * NB + nb, 0, 0)),
        ],
        out_shape=[
            jax.ShapeDtypeStruct((B * NB, 1, BN), jnp.float32),
            jax.ShapeDtypeStruct((B * NB, 1, BN), jnp.int32),
        ],
    )(x, x_dst)

    scat, gath, spos = pl.pallas_call(
        _select_body,
        grid=(B,),
        in_specs=[
            pl.BlockSpec((1, H, W), lambda b: (b, 0, 0)),
            pl.BlockSpec((1, H, W), lambda b: (b, 0, 0)),
        ],
        out_specs=[
            pl.BlockSpec((1, H, W), lambda b: (b, 0, 0)),
            pl.BlockSpec((1, H, W), lambda b: (b, 0, 0)),
            pl.BlockSpec((1, H, W), lambda b: (b, 0, 0)),
        ],
        out_shape=[
            jax.ShapeDtypeStruct((B, H, W), jnp.int32),
            jax.ShapeDtypeStruct((B, H, W), jnp.int32),
            jax.ShapeDtypeStruct((B, H, W), jnp.int32),
        ],
    )(nmax.reshape(B, H, W), nidx.reshape(B, H, W))

    xdp = jnp.zeros((B, NDP, C), x.dtype).at[:, :ND].set(x_dst)

    dm = pl.pallas_call(
        _scatter_body,
        grid=(B, NB),
        in_specs=[
            pl.BlockSpec((1, BN, C), lambda b, nb: (b, nb, 0)),
            pl.BlockSpec((1, 1, BN), lambda b, nb: (b * NB + nb, 0, 0)),
            pl.BlockSpec((1, NDP, C), lambda b, nb: (b, 0, 0)),
        ],
        out_specs=pl.BlockSpec((1, NDP, C), lambda b, nb: (b, 0, 0)),
        out_shape=jax.ShapeDtypeStruct((B, NDP, C), jnp.float32),
        scratch_shapes=[
            pltpu.VMEM((NDP, C), jnp.float32),
            pltpu.VMEM((NDP, 128), jnp.float32),
        ],
    )(x, scat.reshape(B * NB, 1, BN), xdp)

    outp = _sc_unmerge(x, gath, spos, dm.reshape(B * NDP, C))
    return outp.reshape(B, NP8, C)[:, :N]


# speedup vs baseline: 1.7849x; 1.7849x over previous
import jax
import jax.numpy as jnp
from jax import lax
from jax.experimental import pallas as pl
from jax.experimental.pallas import tpu as pltpu
from jax.experimental.pallas import tpu_sc as plsc

H = 128
W = 128
SY = 4
SX = 4
N = H * W
C = 256
ND = (H // SY) * (W // SX)
NDP = 1152
ZROW = 1024
OUT_N = 16384 + 8
R_MERGE = min(N // 2, N - ND)
BN = 1024
NB = N // BN
NSC = 2
NTILE = 16
TOK_TILE = N // NTILE


def _scores_body(xb_ref, xd_ref, nmax_ref, nidx_ref):
    xb = xb_ref[0]
    xd = xd_ref[0]
    mb = xb / (jnp.sqrt(jnp.sum(xb * xb, axis=1, keepdims=True)) + 1e-6)
    md = xd / (jnp.sqrt(jnp.sum(xd * xd, axis=1, keepdims=True)) + 1e-6)
    s = lax.dot_general(mb, md, (((1,), (1,)), ((), ())))
    nmax = jnp.max(s, axis=1)
    iota = lax.broadcasted_iota(jnp.int32, (BN, ND), 1)
    nidx = jnp.min(jnp.where(s == nmax[:, None], iota, ND), axis=1)
    nmax_ref[0, 0] = nmax
    nidx_ref[0, 0] = nidx.astype(jnp.int32)


def _select_body(nm_ref, ni_ref, sc_ref, gi_ref):
    b = pl.program_id(0)
    nm = nm_ref[0]
    ni = ni_ref[0]
    ri = lax.broadcasted_iota(jnp.int32, (H, W), 0)
    ci = lax.broadcasted_iota(jnp.int32, (H, W), 1)
    is_dst = ((ri % SY) == 0) & ((ci % SX) == 0)
    bits = lax.bitcast_convert_type(nm, jnp.int32)
    v = bits ^ (lax.shift_right_arithmetic(bits, 31) & jnp.int32(0x7FFFFFFF))
    min32 = jnp.int32(-2147483648)
    v = jnp.where(is_dst, min32, v)
    r = jnp.int32(R_MERGE)
    cnt_pos = jnp.sum((v >= 0).astype(jnp.int32))
    bucket_pos = cnt_pos >= r
    in_bucket = ((v >= 0) == bucket_pos) & jnp.logical_not(is_dst)
    key = jnp.where(in_bucket, v & jnp.int32(0x7FFFFFFF), jnp.int32(-1))
    rr = jnp.where(bucket_pos, r, r - cnt_pos)

    def body(k, prefix):
        cand = prefix | lax.shift_left(jnp.int32(1), jnp.int32(30) - k)
        cnt = jnp.sum((key >= cand).astype(jnp.int32))
        return jnp.where(cnt >= rr, cand, prefix)

    t = lax.fori_loop(0, 31, body, jnp.int32(0))
    tv = jnp.where(bucket_pos, t, t | min32)
    not_dst = jnp.logical_not(is_dst)
    gt = (v > tv) & not_dst
    eq = (v == tv) & not_dst
    cnt_gt = jnp.sum(gt.astype(jnp.int32))
    need = (r - cnt_gt).astype(jnp.float32)
    e = eq.astype(jnp.float32)
    tri_incl = (ri <= ci).astype(jnp.float32)
    incl = lax.dot_general(e, tri_incl, (((1,), (0,)), ((), ())),
                           precision=lax.Precision.HIGHEST)
    row_tot = incl[:, W - 1:W]
    tri_strict = (ri > ci).astype(jnp.float32)
    off = lax.dot_general(tri_strict, row_tot, (((1,), (0,)), ((), ())),
                          precision=lax.Precision.HIGHEST)
    rank_excl = incl - e + off
    merged = gt | (eq & (rank_excl < need))
    sc_ref[0] = jnp.where(merged, ni, jnp.int32(ND))
    dbin = (ri // SY) * (W // SX) + (ci // SX)
    g = jnp.where(is_dst, dbin, jnp.where(merged, ni, jnp.int32(-1)))
    sel = g >= 0
    t = ri * W + ci
    gi_ref[0] = jnp.where(sel, b * NDP + g,
                          jnp.int32(NSC * NDP) + b * N + t)


def _scatter_body(xb_ref, si_ref, xdp_ref, dm_ref, acc_ref, cnt_ref):
    nb = pl.program_id(1)

    @pl.when(nb == 0)
    def _():
        acc_ref[...] = jnp.zeros_like(acc_ref)
        cnt_ref[...] = jnp.zeros_like(cnt_ref)

    xb = xb_ref[0]
    si = jnp.broadcast_to(si_ref[0], (ND, BN))
    iota = lax.broadcasted_iota(jnp.int32, (ND, BN), 0)
    oht = (si == iota).astype(jnp.float32)
    acc_ref[pl.ds(0, ND), :] += lax.dot_general(
        oht, xb, (((1,), (0,)), ((), ())), precision=lax.Precision.HIGHEST)
    cnt_ref[pl.ds(0, ND), :] += lax.dot_general(
        oht, jnp.ones((BN, 128), jnp.float32), (((1,), (0,)), ((), ())),
        precision=lax.Precision.HIGHEST)

    @pl.when(nb == NB - 1)
    def _():
        dm_ref[0] = (xdp_ref[0] + acc_ref[...]) / (1.0 + cnt_ref[:, 0:1])


def _sc_unmerge_body(gi_hbm, tab_hbm, out_hbm,
                     gb0, gb1, gixbuf, gs0, gs1, ws0, ws1):
    c = lax.axis_index("c")
    s = lax.axis_index("s")
    pltpu.sync_copy(gi_hbm.at[c, pl.ds(s * 8, 8)], gixbuf)

    base = s * TOK_TILE
    gb = (gb0, gb1)
    gsem = (gs0, gs1)
    wsem = (ws0, ws1)
    gld = [None, None]
    wr = [None, None]
    gld[0] = pltpu.async_copy(tab_hbm.at[gixbuf.at[0]], gb0, gs0)
    for k in range(8):
        cur = k % 2
        if k < 7:
            if wr[1 - cur] is not None:
                wr[1 - cur].wait()
            gld[1 - cur] = pltpu.async_copy(
                tab_hbm.at[gixbuf.at[k + 1]], gb[1 - cur], gsem[1 - cur])
        gld[cur].wait()
        wr[cur] = pltpu.async_copy(
            gb[cur], out_hbm.at[c, pl.ds(base + k * 128, 128)], wsem[cur])
    wr[0].wait()
    wr[1].wait()


_sc_unmerge = pl.kernel(
    _sc_unmerge_body,
    out_type=jax.ShapeDtypeStruct((NSC, N, C), jnp.float32),
    mesh=plsc.VectorSubcoreMesh(core_axis_name="c", subcore_axis_name="s"),
    scratch_types=[
        pltpu.VMEM((128, C), jnp.float32),
        pltpu.VMEM((128, C), jnp.float32),
        pltpu.VMEM((8, 128), jnp.int32),
        pltpu.SemaphoreType.DMA,
        pltpu.SemaphoreType.DMA,
        pltpu.SemaphoreType.DMA,
        pltpu.SemaphoreType.DMA,
    ],
)


def kernel(x):
    B = x.shape[0]
    x_dst = x.reshape(B, H // SY, SY, W // SX, SX, C)[:, :, 0, :, 0, :]
    x_dst = x_dst.reshape(B, ND, C)

    nmax, nidx = pl.pallas_call(
        _scores_body,
        grid=(B, NB),
        in_specs=[
            pl.BlockSpec((1, BN, C), lambda b, nb: (b, nb, 0)),
            pl.BlockSpec((1, ND, C), lambda b, nb: (b, 0, 0)),
        ],
        out_specs=[
            pl.BlockSpec((1, 1, BN), lambda b, nb: (b * NB + nb, 0, 0)),
            pl.BlockSpec((1, 1, BN), lambda b, nb: (b * NB + nb, 0, 0)),
        ],
        out_shape=[
            jax.ShapeDtypeStruct((B * NB, 1, BN), jnp.float32),
            jax.ShapeDtypeStruct((B * NB, 1, BN), jnp.int32),
        ],
    )(x, x_dst)

    scat, gath = pl.pallas_call(
        _select_body,
        grid=(B,),
        in_specs=[
            pl.BlockSpec((1, H, W), lambda b: (b, 0, 0)),
            pl.BlockSpec((1, H, W), lambda b: (b, 0, 0)),
        ],
        out_specs=[
            pl.BlockSpec((1, H, W), lambda b: (b, 0, 0)),
            pl.BlockSpec((1, H, W), lambda b: (b, 0, 0)),
        ],
        out_shape=[
            jax.ShapeDtypeStruct((B, H, W), jnp.int32),
            jax.ShapeDtypeStruct((B, H, W), jnp.int32),
        ],
    )(nmax.reshape(B, H, W), nidx.reshape(B, H, W))

    xdp = jnp.zeros((B, NDP, C), x.dtype).at[:, :ND].set(x_dst)

    dm = pl.pallas_call(
        _scatter_body,
        grid=(B, NB),
        in_specs=[
            pl.BlockSpec((1, BN, C), lambda b, nb: (b, nb, 0)),
            pl.BlockSpec((1, 1, BN), lambda b, nb: (b * NB + nb, 0, 0)),
            pl.BlockSpec((1, NDP, C), lambda b, nb: (b, 0, 0)),
        ],
        out_specs=pl.BlockSpec((1, NDP, C), lambda b, nb: (b, 0, 0)),
        out_shape=jax.ShapeDtypeStruct((B, NDP, C), jnp.float32),
        scratch_shapes=[
            pltpu.VMEM((NDP, C), jnp.float32),
            pltpu.VMEM((NDP, 128), jnp.float32),
        ],
    )(x, scat.reshape(B * NB, 1, BN), xdp)

    tab = jnp.concatenate([dm.reshape(B * NDP, C), x.reshape(B * N, C)], 0)
    return _sc_unmerge(gath, tab)


# speedup vs baseline: 2.7924x; 1.5644x over previous
import jax
import jax.numpy as jnp
from jax import lax
from jax.experimental import pallas as pl
from jax.experimental.pallas import tpu as pltpu
from jax.experimental.pallas import tpu_sc as plsc

H = 128
W = 128
SY = 4
SX = 4
N = H * W
C = 256
ND = (H // SY) * (W // SX)
NDP = 1152
ZROW = 1024
OUT_N = 16384 + 8
R_MERGE = min(N // 2, N - ND)
BN = 1024
NB = N // BN
NSC = 2
NTILE = 16
TOK_TILE = N // NTILE


def _scores_body(xb_ref, xd_ref, nmax_ref, nidx_ref):
    xb = xb_ref[0]
    xd = xd_ref[0]
    mb = xb / (jnp.sqrt(jnp.sum(xb * xb, axis=1, keepdims=True)) + 1e-6)
    md = xd / (jnp.sqrt(jnp.sum(xd * xd, axis=1, keepdims=True)) + 1e-6)
    s = lax.dot_general(mb, md, (((1,), (1,)), ((), ())))
    nmax = jnp.max(s, axis=1)
    iota = lax.broadcasted_iota(jnp.int32, (BN, ND), 1)
    nidx = jnp.min(jnp.where(s == nmax[:, None], iota, ND), axis=1)
    nmax_ref[0, 0] = nmax
    nidx_ref[0, 0] = nidx.astype(jnp.int32)


def _select_body(nm_ref, ni_ref, sc_ref, gi_ref):
    b = pl.program_id(0)
    nm = nm_ref[0]
    ni = ni_ref[0]
    ri = lax.broadcasted_iota(jnp.int32, (H, W), 0)
    ci = lax.broadcasted_iota(jnp.int32, (H, W), 1)
    is_dst = ((ri % SY) == 0) & ((ci % SX) == 0)
    bits = lax.bitcast_convert_type(nm, jnp.int32)
    v = bits ^ (lax.shift_right_arithmetic(bits, 31) & jnp.int32(0x7FFFFFFF))
    min32 = jnp.int32(-2147483648)
    v = jnp.where(is_dst, min32, v)
    r = jnp.int32(R_MERGE)
    cnt_pos = jnp.sum((v >= 0).astype(jnp.int32))
    bucket_pos = cnt_pos >= r
    in_bucket = ((v >= 0) == bucket_pos) & jnp.logical_not(is_dst)
    key = jnp.where(in_bucket, v & jnp.int32(0x7FFFFFFF), jnp.int32(-1))
    rr = jnp.where(bucket_pos, r, r - cnt_pos)

    def body(k, prefix):
        cand = prefix | lax.shift_left(jnp.int32(1), jnp.int32(30) - k)
        cnt = jnp.sum((key >= cand).astype(jnp.int32))
        return jnp.where(cnt >= rr, cand, prefix)

    t = lax.fori_loop(0, 31, body, jnp.int32(0))
    tv = jnp.where(bucket_pos, t, t | min32)
    not_dst = jnp.logical_not(is_dst)
    gt = (v > tv) & not_dst
    eq = (v == tv) & not_dst
    cnt_gt = jnp.sum(gt.astype(jnp.int32))
    need = (r - cnt_gt).astype(jnp.float32)
    e = eq.astype(jnp.float32)
    tri_incl = (ri <= ci).astype(jnp.float32)
    incl = lax.dot_general(e, tri_incl, (((1,), (0,)), ((), ())),
                           precision=lax.Precision.HIGHEST)
    row_tot = incl[:, W - 1:W]
    tri_strict = (ri > ci).astype(jnp.float32)
    off = lax.dot_general(tri_strict, row_tot, (((1,), (0,)), ((), ())),
                          precision=lax.Precision.HIGHEST)
    rank_excl = incl - e + off
    merged = gt | (eq & (rank_excl < need))
    sc_ref[0] = jnp.where(merged, ni, jnp.int32(ND))
    dbin = (ri // SY) * (W // SX) + (ci // SX)
    g = jnp.where(is_dst, dbin, jnp.where(merged, ni, jnp.int32(-1)))
    sel = g >= 0
    t = ri * W + ci
    gi_ref[0] = jnp.where(sel, b * NDP + g,
                          jnp.int32(NSC * NDP) + b * N + t)


def _scatter_body(xb_ref, si_ref, xdp_ref, dm_ref, acc_ref, cnt_ref):
    nb = pl.program_id(1)

    @pl.when(nb == 0)
    def _():
        acc_ref[...] = jnp.zeros_like(acc_ref)
        cnt_ref[...] = jnp.zeros_like(cnt_ref)

    xb = xb_ref[0]
    si = jnp.broadcast_to(si_ref[0], (ND, BN))
    iota = lax.broadcasted_iota(jnp.int32, (ND, BN), 0)
    oht = (si == iota).astype(jnp.float32)
    acc_ref[pl.ds(0, ND), :] += lax.dot_general(
        oht, xb, (((1,), (0,)), ((), ())))
    cnt_ref[pl.ds(0, ND), :] += lax.dot_general(
        oht, jnp.ones((BN, 128), jnp.float32), (((1,), (0,)), ((), ())))

    @pl.when(nb == NB - 1)
    def _():
        dm_ref[0] = (xdp_ref[0] + acc_ref[...]) / (1.0 + cnt_ref[:, 0:1])


def _sc_unmerge_body(gi_hbm, tab_hbm, out_hbm,
                     gb0, gb1, gixbuf, gs0, gs1, ws0, ws1):
    c = lax.axis_index("c")
    s = lax.axis_index("s")
    pltpu.sync_copy(gi_hbm.at[c, pl.ds(s * 8, 8)], gixbuf)

    base = s * TOK_TILE
    gb = (gb0, gb1)
    gsem = (gs0, gs1)
    wsem = (ws0, ws1)
    gld = [None, None]
    wr = [None, None]
    gld[0] = pltpu.async_copy(tab_hbm.at[gixbuf.at[0]], gb0, gs0)
    for k in range(8):
        cur = k % 2
        if k < 7:
            if wr[1 - cur] is not None:
                wr[1 - cur].wait()
            gld[1 - cur] = pltpu.async_copy(
                tab_hbm.at[gixbuf.at[k + 1]], gb[1 - cur], gsem[1 - cur])
        gld[cur].wait()
        wr[cur] = pltpu.async_copy(
            gb[cur], out_hbm.at[c, pl.ds(base + k * 128, 128)], wsem[cur])
    wr[0].wait()
    wr[1].wait()


_sc_unmerge = pl.kernel(
    _sc_unmerge_body,
    out_type=jax.ShapeDtypeStruct((NSC, N, C), jnp.float32),
    mesh=plsc.VectorSubcoreMesh(core_axis_name="c", subcore_axis_name="s"),
    scratch_types=[
        pltpu.VMEM((128, C), jnp.float32),
        pltpu.VMEM((128, C), jnp.float32),
        pltpu.VMEM((8, 128), jnp.int32),
        pltpu.SemaphoreType.DMA,
        pltpu.SemaphoreType.DMA,
        pltpu.SemaphoreType.DMA,
        pltpu.SemaphoreType.DMA,
    ],
)


def kernel(x):
    B = x.shape[0]
    x_dst = x.reshape(B, H // SY, SY, W // SX, SX, C)[:, :, 0, :, 0, :]
    x_dst = x_dst.reshape(B, ND, C)

    nmax, nidx = pl.pallas_call(
        _scores_body,
        grid=(B, NB),
        in_specs=[
            pl.BlockSpec((1, BN, C), lambda b, nb: (b, nb, 0)),
            pl.BlockSpec((1, ND, C), lambda b, nb: (b, 0, 0)),
        ],
        out_specs=[
            pl.BlockSpec((1, 1, BN), lambda b, nb: (b * NB + nb, 0, 0)),
            pl.BlockSpec((1, 1, BN), lambda b, nb: (b * NB + nb, 0, 0)),
        ],
        out_shape=[
            jax.ShapeDtypeStruct((B * NB, 1, BN), jnp.float32),
            jax.ShapeDtypeStruct((B * NB, 1, BN), jnp.int32),
        ],
    )(x, x_dst)

    scat, gath = pl.pallas_call(
        _select_body,
        grid=(B,),
        in_specs=[
            pl.BlockSpec((1, H, W), lambda b: (b, 0, 0)),
            pl.BlockSpec((1, H, W), lambda b: (b, 0, 0)),
        ],
        out_specs=[
            pl.BlockSpec((1, H, W), lambda b: (b, 0, 0)),
            pl.BlockSpec((1, H, W), lambda b: (b, 0, 0)),
        ],
        out_shape=[
            jax.ShapeDtypeStruct((B, H, W), jnp.int32),
            jax.ShapeDtypeStruct((B, H, W), jnp.int32),
        ],
    )(nmax.reshape(B, H, W), nidx.reshape(B, H, W))

    xdp = jnp.zeros((B, NDP, C), x.dtype).at[:, :ND].set(x_dst)

    dm = pl.pallas_call(
        _scatter_body,
        grid=(B, NB),
        in_specs=[
            pl.BlockSpec((1, BN, C), lambda b, nb: (b, nb, 0)),
            pl.BlockSpec((1, 1, BN), lambda b, nb: (b * NB + nb, 0, 0)),
            pl.BlockSpec((1, NDP, C), lambda b, nb: (b, 0, 0)),
        ],
        out_specs=pl.BlockSpec((1, NDP, C), lambda b, nb: (b, 0, 0)),
        out_shape=jax.ShapeDtypeStruct((B, NDP, C), jnp.float32),
        scratch_shapes=[
            pltpu.VMEM((NDP, C), jnp.float32),
            pltpu.VMEM((NDP, 128), jnp.float32),
        ],
    )(x, scat.reshape(B * NB, 1, BN), xdp)

    tab = jnp.concatenate([dm.reshape(B * NDP, C), x.reshape(B * N, C)], 0)
    return _sc_unmerge(gath, tab)


# speedup vs baseline: 4.0799x; 1.4611x over previous
import jax
import jax.numpy as jnp
from jax import lax
from jax.experimental import pallas as pl
from jax.experimental.pallas import tpu as pltpu
from jax.experimental.pallas import tpu_sc as plsc

H = 128
W = 128
SY = 4
SX = 4
N = H * W
C = 256
ND = (H // SY) * (W // SX)
NDP = 1152
ZROW = 1024
OUT_N = 16384 + 8
R_MERGE = min(N // 2, N - ND)
BN = 1024
NB = N // BN
NSC = 2
NTILE = 16
TOK_TILE = N // NTILE


def _scores_body(xb_ref, xd_ref, nmax_ref, nidx_ref):
    xb = xb_ref[0]
    xd = xd_ref[0]
    mb = xb / (jnp.sqrt(jnp.sum(xb * xb, axis=1, keepdims=True)) + 1e-6)
    md = xd / (jnp.sqrt(jnp.sum(xd * xd, axis=1, keepdims=True)) + 1e-6)
    s = lax.dot_general(md, mb, (((1,), (1,)), ((), ())))
    nmax = jnp.max(s, axis=0)
    nidx = jnp.argmax(s, axis=0)
    nmax_ref[0, 0] = nmax
    nidx_ref[0, 0] = nidx.astype(jnp.int32)


def _select_body(nm_ref, ni_ref, sc_ref, gi_ref):
    b = pl.program_id(0)
    nm = nm_ref[0]
    ni = ni_ref[0]
    ri = lax.broadcasted_iota(jnp.int32, (H, W), 0)
    ci = lax.broadcasted_iota(jnp.int32, (H, W), 1)
    is_dst = ((ri % SY) == 0) & ((ci % SX) == 0)
    bits = lax.bitcast_convert_type(nm, jnp.int32)
    v = bits ^ (lax.shift_right_arithmetic(bits, 31) & jnp.int32(0x7FFFFFFF))
    min32 = jnp.int32(-2147483648)
    v = jnp.where(is_dst, min32, v)
    r = jnp.int32(R_MERGE)
    cnt_pos = jnp.sum((v >= 0).astype(jnp.int32))
    bucket_pos = cnt_pos >= r
    in_bucket = ((v >= 0) == bucket_pos) & jnp.logical_not(is_dst)
    key = jnp.where(in_bucket, v & jnp.int32(0x7FFFFFFF), jnp.int32(-1))
    rr = jnp.where(bucket_pos, r, r - cnt_pos)

    def body(k, prefix):
        cand = prefix | lax.shift_left(jnp.int32(1), jnp.int32(30) - k)
        cnt = jnp.sum((key >= cand).astype(jnp.int32))
        return jnp.where(cnt >= rr, cand, prefix)

    t = lax.fori_loop(0, 31, body, jnp.int32(0))
    tv = jnp.where(bucket_pos, t, t | min32)
    not_dst = jnp.logical_not(is_dst)
    gt = (v > tv) & not_dst
    eq = (v == tv) & not_dst
    cnt_gt = jnp.sum(gt.astype(jnp.int32))
    need = (r - cnt_gt).astype(jnp.float32)
    e = eq.astype(jnp.float32)
    tri_incl = (ri <= ci).astype(jnp.float32)
    incl = lax.dot_general(e, tri_incl, (((1,), (0,)), ((), ())),
                           precision=lax.Precision.HIGHEST)
    row_tot = incl[:, W - 1:W]
    tri_strict = (ri > ci).astype(jnp.float32)
    off = lax.dot_general(tri_strict, row_tot, (((1,), (0,)), ((), ())),
                          precision=lax.Precision.HIGHEST)
    rank_excl = incl - e + off
    merged = gt | (eq & (rank_excl < need))
    sc_ref[0] = jnp.where(merged, ni, jnp.int32(ND))
    dbin = (ri // SY) * (W // SX) + (ci // SX)
    g = jnp.where(is_dst, dbin, jnp.where(merged, ni, jnp.int32(-1)))
    sel = g >= 0
    t = ri * W + ci
    gi_ref[0] = jnp.where(sel, b * NDP + g,
                          jnp.int32(NSC * NDP) + b * N + t)


def _scatter_body(xb_ref, si_ref, xdp_ref, dm_ref, acc_ref, cnt_ref):
    nb = pl.program_id(1)

    @pl.when(nb == 0)
    def _():
        acc_ref[...] = jnp.zeros_like(acc_ref)
        cnt_ref[...] = jnp.zeros_like(cnt_ref)

    xb = xb_ref[0]
    si = jnp.broadcast_to(si_ref[0], (ND, BN))
    iota = lax.broadcasted_iota(jnp.int32, (ND, BN), 0)
    oht = (si == iota).astype(jnp.float32)
    acc_ref[pl.ds(0, ND), :] += lax.dot_general(
        oht, xb, (((1,), (0,)), ((), ())))
    cnt_ref[pl.ds(0, ND), :] += lax.dot_general(
        oht, jnp.ones((BN, 128), jnp.float32), (((1,), (0,)), ((), ())))

    @pl.when(nb == NB - 1)
    def _():
        dm_ref[0] = (xdp_ref[0] + acc_ref[...]) / (1.0 + cnt_ref[:, 0:1])


def _sc_unmerge_body(gi_hbm, tab_hbm, out_hbm,
                     gb0, gb1, gixbuf, gs0, gs1, ws0, ws1):
    c = lax.axis_index("c")
    s = lax.axis_index("s")
    pltpu.sync_copy(gi_hbm.at[c, pl.ds(s * 8, 8)], gixbuf)

    base = s * TOK_TILE
    gb = (gb0, gb1)
    gsem = (gs0, gs1)
    wsem = (ws0, ws1)
    gld = [None, None]
    wr = [None, None]
    gld[0] = pltpu.async_copy(tab_hbm.at[gixbuf.at[0]], gb0, gs0)
    for k in range(8):
        cur = k % 2
        if k < 7:
            if wr[1 - cur] is not None:
                wr[1 - cur].wait()
            gld[1 - cur] = pltpu.async_copy(
                tab_hbm.at[gixbuf.at[k + 1]], gb[1 - cur], gsem[1 - cur])
        gld[cur].wait()
        wr[cur] = pltpu.async_copy(
            gb[cur], out_hbm.at[c, pl.ds(base + k * 128, 128)], wsem[cur])
    wr[0].wait()
    wr[1].wait()


_sc_unmerge = pl.kernel(
    _sc_unmerge_body,
    out_type=jax.ShapeDtypeStruct((NSC, N, C), jnp.float32),
    mesh=plsc.VectorSubcoreMesh(core_axis_name="c", subcore_axis_name="s"),
    scratch_types=[
        pltpu.VMEM((128, C), jnp.float32),
        pltpu.VMEM((128, C), jnp.float32),
        pltpu.VMEM((8, 128), jnp.int32),
        pltpu.SemaphoreType.DMA,
        pltpu.SemaphoreType.DMA,
        pltpu.SemaphoreType.DMA,
        pltpu.SemaphoreType.DMA,
    ],
)


def kernel(x):
    B = x.shape[0]
    x_dst = x.reshape(B, H // SY, SY, W // SX, SX, C)[:, :, 0, :, 0, :]
    x_dst = x_dst.reshape(B, ND, C)

    nmax, nidx = pl.pallas_call(
        _scores_body,
        grid=(B, NB),
        in_specs=[
            pl.BlockSpec((1, BN, C), lambda b, nb: (b, nb, 0)),
            pl.BlockSpec((1, ND, C), lambda b, nb: (b, 0, 0)),
        ],
        out_specs=[
            pl.BlockSpec((1, 1, BN), lambda b, nb: (b * NB + nb, 0, 0)),
            pl.BlockSpec((1, 1, BN), lambda b, nb: (b * NB + nb, 0, 0)),
        ],
        out_shape=[
            jax.ShapeDtypeStruct((B * NB, 1, BN), jnp.float32),
            jax.ShapeDtypeStruct((B * NB, 1, BN), jnp.int32),
        ],
    )(x, x_dst)

    scat, gath = pl.pallas_call(
        _select_body,
        grid=(B,),
        in_specs=[
            pl.BlockSpec((1, H, W), lambda b: (b, 0, 0)),
            pl.BlockSpec((1, H, W), lambda b: (b, 0, 0)),
        ],
        out_specs=[
            pl.BlockSpec((1, H, W), lambda b: (b, 0, 0)),
            pl.BlockSpec((1, H, W), lambda b: (b, 0, 0)),
        ],
        out_shape=[
            jax.ShapeDtypeStruct((B, H, W), jnp.int32),
            jax.ShapeDtypeStruct((B, H, W), jnp.int32),
        ],
    )(nmax.reshape(B, H, W), nidx.reshape(B, H, W))

    xdp = jnp.zeros((B, NDP, C), x.dtype).at[:, :ND].set(x_dst)

    dm = pl.pallas_call(
        _scatter_body,
        grid=(B, NB),
        in_specs=[
            pl.BlockSpec((1, BN, C), lambda b, nb: (b, nb, 0)),
            pl.BlockSpec((1, 1, BN), lambda b, nb: (b * NB + nb, 0, 0)),
            pl.BlockSpec((1, NDP, C), lambda b, nb: (b, 0, 0)),
        ],
        out_specs=pl.BlockSpec((1, NDP, C), lambda b, nb: (b, 0, 0)),
        out_shape=jax.ShapeDtypeStruct((B, NDP, C), jnp.float32),
        scratch_shapes=[
            pltpu.VMEM((NDP, C), jnp.float32),
            pltpu.VMEM((NDP, 128), jnp.float32),
        ],
    )(x, scat.reshape(B * NB, 1, BN), xdp)

    tab = jnp.concatenate([dm.reshape(B * NDP, C), x.reshape(B * N, C)], 0)
    return _sc_unmerge(gath, tab)


# speedup vs baseline: 4.1401x; 1.0147x over previous
import jax
import jax.numpy as jnp
from jax import lax
from jax.experimental import pallas as pl
from jax.experimental.pallas import tpu as pltpu
from jax.experimental.pallas import tpu_sc as plsc

H = 128
W = 128
SY = 4
SX = 4
N = H * W
C = 256
ND = (H // SY) * (W // SX)
NDP = 1152
ZROW = 1024
OUT_N = 16384 + 8
R_MERGE = min(N // 2, N - ND)
BN = 1024
NB = N // BN
NSC = 2
NTILE = 16
TOK_TILE = N // NTILE


def _scores_body(xb_ref, xd_ref, sc_ref, gi_ref, nm_s, ni_s):
    nb = pl.program_id(1)
    xb = xb_ref[0]
    xd = xd_ref[0]
    mb = xb / (jnp.sqrt(jnp.sum(xb * xb, axis=1, keepdims=True)) + 1e-6)
    md = xd / (jnp.sqrt(jnp.sum(xd * xd, axis=1, keepdims=True)) + 1e-6)
    s = lax.dot_general(md, mb, (((1,), (1,)), ((), ())))
    nmax = jnp.max(s, axis=0)
    nidx = jnp.argmax(s, axis=0)
    nm_s[pl.ds(nb * (BN // W), BN // W), :] = nmax.reshape(BN // W, W)
    ni_s[pl.ds(nb * (BN // W), BN // W), :] = nidx.astype(jnp.int32).reshape(
        BN // W, W)

    @pl.when(nb == NB - 1)
    def _():
        _select(nm_s[...], ni_s[...], sc_ref, gi_ref)


def _select(nm, ni, sc_ref, gi_ref):
    b = pl.program_id(0)
    ri = lax.broadcasted_iota(jnp.int32, (H, W), 0)
    ci = lax.broadcasted_iota(jnp.int32, (H, W), 1)
    is_dst = ((ri % SY) == 0) & ((ci % SX) == 0)
    bits = lax.bitcast_convert_type(nm, jnp.int32)
    v = bits ^ (lax.shift_right_arithmetic(bits, 31) & jnp.int32(0x7FFFFFFF))
    min32 = jnp.int32(-2147483648)
    v = jnp.where(is_dst, min32, v)
    r = jnp.int32(R_MERGE)
    cnt_pos = jnp.sum((v >= 0).astype(jnp.int32))
    bucket_pos = cnt_pos >= r
    in_bucket = ((v >= 0) == bucket_pos) & jnp.logical_not(is_dst)
    key = jnp.where(in_bucket, v & jnp.int32(0x7FFFFFFF), jnp.int32(-1))
    rr = jnp.where(bucket_pos, r, r - cnt_pos)

    def body(k, prefix):
        cand = prefix | lax.shift_left(jnp.int32(1), jnp.int32(30) - k)
        cnt = jnp.sum((key >= cand).astype(jnp.int32))
        return jnp.where(cnt >= rr, cand, prefix)

    t = lax.fori_loop(0, 31, body, jnp.int32(0))
    tv = jnp.where(bucket_pos, t, t | min32)
    not_dst = jnp.logical_not(is_dst)
    gt = (v > tv) & not_dst
    eq = (v == tv) & not_dst
    cnt_gt = jnp.sum(gt.astype(jnp.int32))
    need = (r - cnt_gt).astype(jnp.float32)
    e = eq.astype(jnp.float32)
    tri_incl = (ri <= ci).astype(jnp.float32)
    incl = lax.dot_general(e, tri_incl, (((1,), (0,)), ((), ())),
                           precision=lax.Precision.HIGHEST)
    row_tot = incl[:, W - 1:W]
    tri_strict = (ri > ci).astype(jnp.float32)
    off = lax.dot_general(tri_strict, row_tot, (((1,), (0,)), ((), ())),
                          precision=lax.Precision.HIGHEST)
    rank_excl = incl - e + off
    merged = gt | (eq & (rank_excl < need))
    sc_ref[0] = jnp.where(merged, ni, jnp.int32(ND))
    dbin = (ri // SY) * (W // SX) + (ci // SX)
    g = jnp.where(is_dst, dbin, jnp.where(merged, ni, jnp.int32(-1)))
    sel = g >= 0
    t = ri * W + ci
    gi_ref[0] = jnp.where(sel, b * NDP + g,
                          jnp.int32(NSC * NDP) + b * N + t)


def _scatter_body(xb_ref, si_ref, xdp_ref, dm_ref, acc_ref, cnt_ref):
    nb = pl.program_id(1)

    @pl.when(nb == 0)
    def _():
        acc_ref[...] = jnp.zeros_like(acc_ref)
        cnt_ref[...] = jnp.zeros_like(cnt_ref)

    xb = xb_ref[0]
    si = jnp.broadcast_to(si_ref[0], (ND, BN))
    iota = lax.broadcasted_iota(jnp.int32, (ND, BN), 0)
    oht = (si == iota).astype(jnp.float32)
    acc_ref[pl.ds(0, ND), :] += lax.dot_general(
        oht, xb, (((1,), (0,)), ((), ())))
    cnt_ref[pl.ds(0, ND), :] += lax.dot_general(
        oht, jnp.ones((BN, 128), jnp.float32), (((1,), (0,)), ((), ())))

    @pl.when(nb == NB - 1)
    def _():
        dm_ref[0] = (xdp_ref[0] + acc_ref[...]) / (1.0 + cnt_ref[:, 0:1])


def _sc_unmerge_body(gi_hbm, tab_hbm, out_hbm,
                     gb0, gb1, gixbuf, gs0, gs1, ws0, ws1):
    c = lax.axis_index("c")
    s = lax.axis_index("s")
    pltpu.sync_copy(gi_hbm.at[c, pl.ds(s * 8, 8)], gixbuf)

    base = s * TOK_TILE
    gb = (gb0, gb1)
    gsem = (gs0, gs1)
    wsem = (ws0, ws1)
    gld = [None, None]
    wr = [None, None]
    gld[0] = pltpu.async_copy(tab_hbm.at[gixbuf.at[0]], gb0, gs0)
    for k in range(8):
        cur = k % 2
        if k < 7:
            if wr[1 - cur] is not None:
                wr[1 - cur].wait()
            gld[1 - cur] = pltpu.async_copy(
                tab_hbm.at[gixbuf.at[k + 1]], gb[1 - cur], gsem[1 - cur])
        gld[cur].wait()
        wr[cur] = pltpu.async_copy(
            gb[cur], out_hbm.at[c, pl.ds(base + k * 128, 128)], wsem[cur])
    wr[0].wait()
    wr[1].wait()


_sc_unmerge = pl.kernel(
    _sc_unmerge_body,
    out_type=jax.ShapeDtypeStruct((NSC, N, C), jnp.float32),
    mesh=plsc.VectorSubcoreMesh(core_axis_name="c", subcore_axis_name="s"),
    scratch_types=[
        pltpu.VMEM((128, C), jnp.float32),
        pltpu.VMEM((128, C), jnp.float32),
        pltpu.VMEM((8, 128), jnp.int32),
        pltpu.SemaphoreType.DMA,
        pltpu.SemaphoreType.DMA,
        pltpu.SemaphoreType.DMA,
        pltpu.SemaphoreType.DMA,
    ],
)


def kernel(x):
    B = x.shape[0]
    x_dst = x.reshape(B, H // SY, SY, W // SX, SX, C)[:, :, 0, :, 0, :]
    x_dst = x_dst.reshape(B, ND, C)

    scat, gath = pl.pallas_call(
        _scores_body,
        grid=(B, NB),
        in_specs=[
            pl.BlockSpec((1, BN, C), lambda b, nb: (b, nb, 0)),
            pl.BlockSpec((1, ND, C), lambda b, nb: (b, 0, 0)),
        ],
        out_specs=[
            pl.BlockSpec((1, H, W), lambda b, nb: (b, 0, 0)),
            pl.BlockSpec((1, H, W), lambda b, nb: (b, 0, 0)),
        ],
        out_shape=[
            jax.ShapeDtypeStruct((B, H, W), jnp.int32),
            jax.ShapeDtypeStruct((B, H, W), jnp.int32),
        ],
        scratch_shapes=[
            pltpu.VMEM((H, W), jnp.float32),
            pltpu.VMEM((H, W), jnp.int32),
        ],
    )(x, x_dst)

    xdp = jnp.zeros((B, NDP, C), x.dtype).at[:, :ND].set(x_dst)

    dm = pl.pallas_call(
        _scatter_body,
        grid=(B, NB),
        in_specs=[
            pl.BlockSpec((1, BN, C), lambda b, nb: (b, nb, 0)),
            pl.BlockSpec((1, 1, BN), lambda b, nb: (b * NB + nb, 0, 0)),
            pl.BlockSpec((1, NDP, C), lambda b, nb: (b, 0, 0)),
        ],
        out_specs=pl.BlockSpec((1, NDP, C), lambda b, nb: (b, 0, 0)),
        out_shape=jax.ShapeDtypeStruct((B, NDP, C), jnp.float32),
        scratch_shapes=[
            pltpu.VMEM((NDP, C), jnp.float32),
            pltpu.VMEM((NDP, 128), jnp.float32),
        ],
    )(x, scat.reshape(B * NB, 1, BN), xdp)

    tab = jnp.concatenate([dm.reshape(B * NDP, C), x.reshape(B * N, C)], 0)
    return _sc_unmerge(gath, tab)


# speedup vs baseline: 4.2039x; 1.0154x over previous
import jax
import jax.numpy as jnp
from jax import lax
from jax.experimental import pallas as pl
from jax.experimental.pallas import tpu as pltpu
from jax.experimental.pallas import tpu_sc as plsc

H = 128
W = 128
SY = 4
SX = 4
N = H * W
C = 256
ND = (H // SY) * (W // SX)
NDP = 1152
ZROW = 1024
OUT_N = 16384 + 8
R_MERGE = min(N // 2, N - ND)
BN = 1024
NB = N // BN
NSC = 2
NTILE = 16
TOK_TILE = N // NTILE


def _scores_body(xb_ref, xd_ref, sc_ref, gi_ref, nm_s, ni_s):
    nb = pl.program_id(1)
    xb = xb_ref[0]
    xd = xd_ref[0]
    mb = xb / (jnp.sqrt(jnp.sum(xb * xb, axis=1, keepdims=True)) + 1e-6)
    md = xd / (jnp.sqrt(jnp.sum(xd * xd, axis=1, keepdims=True)) + 1e-6)
    s = lax.dot_general(md, mb, (((1,), (1,)), ((), ())))
    nmax = jnp.max(s, axis=0)
    nidx = jnp.argmax(s, axis=0)
    nm_s[pl.ds(nb * (BN // W), BN // W), :] = nmax.reshape(BN // W, W)
    ni_s[pl.ds(nb * (BN // W), BN // W), :] = nidx.astype(jnp.int32).reshape(
        BN // W, W)

    @pl.when(nb == NB - 1)
    def _():
        _select(nm_s[...], ni_s[...], sc_ref, gi_ref)


def _select(nm, ni, sc_ref, gi_ref):
    b = pl.program_id(0)
    ri = lax.broadcasted_iota(jnp.int32, (H, W), 0)
    ci = lax.broadcasted_iota(jnp.int32, (H, W), 1)
    is_dst = ((ri % SY) == 0) & ((ci % SX) == 0)
    bits = lax.bitcast_convert_type(nm, jnp.int32)
    v = bits ^ (lax.shift_right_arithmetic(bits, 31) & jnp.int32(0x7FFFFFFF))
    min32 = jnp.int32(-2147483648)
    v = jnp.where(is_dst, min32, v)
    r = jnp.int32(R_MERGE)
    cnt_pos = jnp.sum((v >= 0).astype(jnp.int32))
    bucket_pos = cnt_pos >= r
    in_bucket = ((v >= 0) == bucket_pos) & jnp.logical_not(is_dst)
    key = jnp.where(in_bucket, v & jnp.int32(0x7FFFFFFF), jnp.int32(-1))
    rr = jnp.where(bucket_pos, r, r - cnt_pos)

    def body(k, prefix):
        cand = prefix | lax.shift_left(jnp.int32(1), jnp.int32(30) - k)
        cnt = jnp.sum((key >= cand).astype(jnp.int32))
        return jnp.where(cnt >= rr, cand, prefix)

    t = lax.fori_loop(0, 31, body, jnp.int32(0))
    tv = jnp.where(bucket_pos, t, t | min32)
    not_dst = jnp.logical_not(is_dst)
    gt = (v > tv) & not_dst
    eq = (v == tv) & not_dst
    cnt_gt = jnp.sum(gt.astype(jnp.int32))
    need = (r - cnt_gt).astype(jnp.float32)
    e = eq.astype(jnp.float32)
    tri_incl = (ri <= ci).astype(jnp.float32)
    incl = lax.dot_general(e, tri_incl, (((1,), (0,)), ((), ())),
                           precision=lax.Precision.HIGHEST)
    row_tot = incl[:, W - 1:W]
    tri_strict = (ri > ci).astype(jnp.float32)
    off = lax.dot_general(tri_strict, row_tot, (((1,), (0,)), ((), ())),
                          precision=lax.Precision.HIGHEST)
    rank_excl = incl - e + off
    merged = gt | (eq & (rank_excl < need))
    dbin = (ri // SY) * (W // SX) + (ci // SX)
    sc_ref[0] = jnp.where(is_dst, dbin, jnp.where(merged, ni, jnp.int32(ND)))
    g = jnp.where(is_dst, dbin, jnp.where(merged, ni, jnp.int32(-1)))
    sel = g >= 0
    t = ri * W + ci
    gi_ref[0] = jnp.where(sel, b * NDP + g,
                          jnp.int32(NSC * NDP) + b * N + t)


def _scatter_body(xb_ref, si_ref, dm_ref, acc_ref, cnt_ref):
    nb = pl.program_id(1)

    @pl.when(nb == 0)
    def _():
        acc_ref[...] = jnp.zeros_like(acc_ref)
        cnt_ref[...] = jnp.zeros_like(cnt_ref)

    xb = xb_ref[0]
    si = jnp.broadcast_to(si_ref[0], (ND, BN))
    iota = lax.broadcasted_iota(jnp.int32, (ND, BN), 0)
    oht = (si == iota).astype(jnp.float32)
    acc_ref[pl.ds(0, ND), :] += lax.dot_general(
        oht, xb, (((1,), (0,)), ((), ())))
    cnt_ref[pl.ds(0, ND), :] += lax.dot_general(
        oht, jnp.ones((BN, 128), jnp.float32), (((1,), (0,)), ((), ())))

    @pl.when(nb == NB - 1)
    def _():
        dm_ref[0] = acc_ref[...] / jnp.maximum(cnt_ref[:, 0:1], 1.0)


def _sc_unmerge_body(gi_hbm, tab_hbm, out_hbm,
                     gb0, gb1, gixbuf, gs0, gs1, ws0, ws1):
    c = lax.axis_index("c")
    s = lax.axis_index("s")
    pltpu.sync_copy(gi_hbm.at[c, pl.ds(s * 8, 8)], gixbuf)

    base = s * TOK_TILE
    gb = (gb0, gb1)
    gsem = (gs0, gs1)
    wsem = (ws0, ws1)
    gld = [None, None]
    wr = [None, None]
    gld[0] = pltpu.async_copy(tab_hbm.at[gixbuf.at[0]], gb0, gs0)
    for k in range(8):
        cur = k % 2
        if k < 7:
            if wr[1 - cur] is not None:
                wr[1 - cur].wait()
            gld[1 - cur] = pltpu.async_copy(
                tab_hbm.at[gixbuf.at[k + 1]], gb[1 - cur], gsem[1 - cur])
        gld[cur].wait()
        wr[cur] = pltpu.async_copy(
            gb[cur], out_hbm.at[c, pl.ds(base + k * 128, 128)], wsem[cur])
    wr[0].wait()
    wr[1].wait()


_sc_unmerge = pl.kernel(
    _sc_unmerge_body,
    out_type=jax.ShapeDtypeStruct((NSC, N, C), jnp.float32),
    mesh=plsc.VectorSubcoreMesh(core_axis_name="c", subcore_axis_name="s"),
    scratch_types=[
        pltpu.VMEM((128, C), jnp.float32),
        pltpu.VMEM((128, C), jnp.float32),
        pltpu.VMEM((8, 128), jnp.int32),
        pltpu.SemaphoreType.DMA,
        pltpu.SemaphoreType.DMA,
        pltpu.SemaphoreType.DMA,
        pltpu.SemaphoreType.DMA,
    ],
)


def kernel(x):
    B = x.shape[0]
    x_dst = x.reshape(B, H // SY, SY, W // SX, SX, C)[:, :, 0, :, 0, :]
    x_dst = x_dst.reshape(B, ND, C)

    scat, gath = pl.pallas_call(
        _scores_body,
        grid=(B, NB),
        in_specs=[
            pl.BlockSpec((1, BN, C), lambda b, nb: (b, nb, 0)),
            pl.BlockSpec((1, ND, C), lambda b, nb: (b, 0, 0)),
        ],
        out_specs=[
            pl.BlockSpec((1, H, W), lambda b, nb: (b, 0, 0)),
            pl.BlockSpec((1, H, W), lambda b, nb: (b, 0, 0)),
        ],
        out_shape=[
            jax.ShapeDtypeStruct((B, H, W), jnp.int32),
            jax.ShapeDtypeStruct((B, H, W), jnp.int32),
        ],
        scratch_shapes=[
            pltpu.VMEM((H, W), jnp.float32),
            pltpu.VMEM((H, W), jnp.int32),
        ],
    )(x, x_dst)

    dm = pl.pallas_call(
        _scatter_body,
        grid=(B, NB),
        in_specs=[
            pl.BlockSpec((1, BN, C), lambda b, nb: (b, nb, 0)),
            pl.BlockSpec((1, 1, BN), lambda b, nb: (b * NB + nb, 0, 0)),
        ],
        out_specs=pl.BlockSpec((1, NDP, C), lambda b, nb: (b, 0, 0)),
        out_shape=jax.ShapeDtypeStruct((B, NDP, C), jnp.float32),
        scratch_shapes=[
            pltpu.VMEM((NDP, C), jnp.float32),
            pltpu.VMEM((NDP, 128), jnp.float32),
        ],
    )(x, scat.reshape(B * NB, 1, BN))

    tab = jnp.concatenate([dm.reshape(B * NDP, C), x.reshape(B * N, C)], 0)
    return _sc_unmerge(gath, tab)


# speedup vs baseline: 4.3268x; 1.0292x over previous
import jax
import jax.numpy as jnp
from jax import lax
from jax.experimental import pallas as pl
from jax.experimental.pallas import tpu as pltpu
from jax.experimental.pallas import tpu_sc as plsc

H = 128
W = 128
SY = 4
SX = 4
N = H * W
C = 256
ND = (H // SY) * (W // SX)
NDP = 1152
ZROW = 1024
OUT_N = 16384 + 8
R_MERGE = min(N // 2, N - ND)
BN = 2048
NB = N // BN
NSC = 2
NTILE = 16
TOK_TILE = N // NTILE


def _scores_body(xb_ref, xd_ref, sc_ref, gi_ref, nm_s, ni_s):
    nb = pl.program_id(1)
    xb = xb_ref[0]
    xd = xd_ref[0]
    mb = xb / (jnp.sqrt(jnp.sum(xb * xb, axis=1, keepdims=True)) + 1e-6)
    md = xd / (jnp.sqrt(jnp.sum(xd * xd, axis=1, keepdims=True)) + 1e-6)
    s = lax.dot_general(md, mb, (((1,), (1,)), ((), ())))
    nmax = jnp.max(s, axis=0)
    nidx = jnp.argmax(s, axis=0)
    nm_s[pl.ds(nb * (BN // W), BN // W), :] = nmax.reshape(BN // W, W)
    ni_s[pl.ds(nb * (BN // W), BN // W), :] = nidx.astype(jnp.int32).reshape(
        BN // W, W)

    @pl.when(nb == NB - 1)
    def _():
        _select(nm_s[...], ni_s[...], sc_ref, gi_ref)


def _select(nm, ni, sc_ref, gi_ref):
    b = pl.program_id(0)
    ri = lax.broadcasted_iota(jnp.int32, (H, W), 0)
    ci = lax.broadcasted_iota(jnp.int32, (H, W), 1)
    is_dst = ((ri % SY) == 0) & ((ci % SX) == 0)
    bits = lax.bitcast_convert_type(nm, jnp.int32)
    v = bits ^ (lax.shift_right_arithmetic(bits, 31) & jnp.int32(0x7FFFFFFF))
    min32 = jnp.int32(-2147483648)
    v = jnp.where(is_dst, min32, v)
    r = jnp.int32(R_MERGE)
    cnt_pos = jnp.sum((v >= 0).astype(jnp.int32))
    bucket_pos = cnt_pos >= r
    in_bucket = ((v >= 0) == bucket_pos) & jnp.logical_not(is_dst)
    key = jnp.where(in_bucket, v & jnp.int32(0x7FFFFFFF), jnp.int32(-1))
    rr = jnp.where(bucket_pos, r, r - cnt_pos)

    def body(k, prefix):
        cand = prefix | lax.shift_left(jnp.int32(1), jnp.int32(30) - k)
        cnt = jnp.sum((key >= cand).astype(jnp.int32))
        return jnp.where(cnt >= rr, cand, prefix)

    t = lax.fori_loop(0, 31, body, jnp.int32(0))
    tv = jnp.where(bucket_pos, t, t | min32)
    not_dst = jnp.logical_not(is_dst)
    gt = (v > tv) & not_dst
    eq = (v == tv) & not_dst
    cnt_gt = jnp.sum(gt.astype(jnp.int32))
    need = (r - cnt_gt).astype(jnp.float32)
    e = eq.astype(jnp.float32)
    tri_incl = (ri <= ci).astype(jnp.float32)
    incl = lax.dot_general(e, tri_incl, (((1,), (0,)), ((), ())),
                           precision=lax.Precision.HIGHEST)
    row_tot = incl[:, W - 1:W]
    tri_strict = (ri > ci).astype(jnp.float32)
    off = lax.dot_general(tri_strict, row_tot, (((1,), (0,)), ((), ())),
                          precision=lax.Precision.HIGHEST)
    rank_excl = incl - e + off
    merged = gt | (eq & (rank_excl < need))
    dbin = (ri // SY) * (W // SX) + (ci // SX)
    sc_ref[0] = jnp.where(is_dst, dbin, jnp.where(merged, ni, jnp.int32(ND)))
    g = jnp.where(is_dst, dbin, jnp.where(merged, ni, jnp.int32(-1)))
    sel = g >= 0
    t = ri * W + ci
    gi_ref[0] = jnp.where(sel, b * NDP + g,
                          jnp.int32(NSC * NDP) + b * N + t)


def _scatter_body(xb_ref, si_ref, dm_ref, acc_ref, cnt_ref):
    nb = pl.program_id(1)

    @pl.when(nb == 0)
    def _():
        acc_ref[...] = jnp.zeros_like(acc_ref)
        cnt_ref[...] = jnp.zeros_like(cnt_ref)

    xb = xb_ref[0]
    si = jnp.broadcast_to(si_ref[0], (ND, BN))
    iota = lax.broadcasted_iota(jnp.int32, (ND, BN), 0)
    oht = (si == iota).astype(jnp.float32)
    acc_ref[pl.ds(0, ND), :] += lax.dot_general(
        oht, xb, (((1,), (0,)), ((), ())))
    cnt_ref[pl.ds(0, ND), :] += lax.dot_general(
        oht, jnp.ones((BN, 128), jnp.float32), (((1,), (0,)), ((), ())))

    @pl.when(nb == NB - 1)
    def _():
        dm_ref[0] = acc_ref[...] / jnp.maximum(cnt_ref[:, 0:1], 1.0)


def _sc_unmerge_body(gi_hbm, tab_hbm, out_hbm,
                     gb0, gb1, gixbuf, gs0, gs1, ws0, ws1):
    c = lax.axis_index("c")
    s = lax.axis_index("s")
    pltpu.sync_copy(gi_hbm.at[c, pl.ds(s * 8, 8)], gixbuf)

    base = s * TOK_TILE
    gb = (gb0, gb1)
    gsem = (gs0, gs1)
    wsem = (ws0, ws1)
    gld = [None, None]
    wr = [None, None]
    gld[0] = pltpu.async_copy(tab_hbm.at[gixbuf.at[0]], gb0, gs0)
    for k in range(8):
        cur = k % 2
        if k < 7:
            if wr[1 - cur] is not None:
                wr[1 - cur].wait()
            gld[1 - cur] = pltpu.async_copy(
                tab_hbm.at[gixbuf.at[k + 1]], gb[1 - cur], gsem[1 - cur])
        gld[cur].wait()
        wr[cur] = pltpu.async_copy(
            gb[cur], out_hbm.at[c, pl.ds(base + k * 128, 128)], wsem[cur])
    wr[0].wait()
    wr[1].wait()


_sc_unmerge = pl.kernel(
    _sc_unmerge_body,
    out_type=jax.ShapeDtypeStruct((NSC, N, C), jnp.float32),
    mesh=plsc.VectorSubcoreMesh(core_axis_name="c", subcore_axis_name="s"),
    scratch_types=[
        pltpu.VMEM((128, C), jnp.float32),
        pltpu.VMEM((128, C), jnp.float32),
        pltpu.VMEM((8, 128), jnp.int32),
        pltpu.SemaphoreType.DMA,
        pltpu.SemaphoreType.DMA,
        pltpu.SemaphoreType.DMA,
        pltpu.SemaphoreType.DMA,
    ],
)


def kernel(x):
    B = x.shape[0]
    x_dst = x.reshape(B, H // SY, SY, W // SX, SX, C)[:, :, 0, :, 0, :]
    x_dst = x_dst.reshape(B, ND, C)

    scat, gath = pl.pallas_call(
        _scores_body,
        grid=(B, NB),
        in_specs=[
            pl.BlockSpec((1, BN, C), lambda b, nb: (b, nb, 0)),
            pl.BlockSpec((1, ND, C), lambda b, nb: (b, 0, 0)),
        ],
        out_specs=[
            pl.BlockSpec((1, H, W), lambda b, nb: (b, 0, 0)),
            pl.BlockSpec((1, H, W), lambda b, nb: (b, 0, 0)),
        ],
        out_shape=[
            jax.ShapeDtypeStruct((B, H, W), jnp.int32),
            jax.ShapeDtypeStruct((B, H, W), jnp.int32),
        ],
        scratch_shapes=[
            pltpu.VMEM((H, W), jnp.float32),
            pltpu.VMEM((H, W), jnp.int32),
        ],
    )(x, x_dst)

    dm = pl.pallas_call(
        _scatter_body,
        grid=(B, NB),
        in_specs=[
            pl.BlockSpec((1, BN, C), lambda b, nb: (b, nb, 0)),
            pl.BlockSpec((1, 1, BN), lambda b, nb: (b * NB + nb, 0, 0)),
        ],
        out_specs=pl.BlockSpec((1, NDP, C), lambda b, nb: (b, 0, 0)),
        out_shape=jax.ShapeDtypeStruct((B, NDP, C), jnp.float32),
        scratch_shapes=[
            pltpu.VMEM((NDP, C), jnp.float32),
            pltpu.VMEM((NDP, 128), jnp.float32),
        ],
    )(x, scat.reshape(B * NB, 1, BN))

    tab = jnp.concatenate([dm.reshape(B * NDP, C), x.reshape(B * N, C)], 0)
    return _sc_unmerge(gath, tab)


# speedup vs baseline: 4.3755x; 1.0113x over previous
import jax
import jax.numpy as jnp
from jax import lax
from jax.experimental import pallas as pl
from jax.experimental.pallas import tpu as pltpu
from jax.experimental.pallas import tpu_sc as plsc

H = 128
W = 128
SY = 4
SX = 4
N = H * W
C = 256
ND = (H // SY) * (W // SX)
NDP = 1152
ZROW = 1024
OUT_N = 16384 + 8
R_MERGE = min(N // 2, N - ND)
BN = 4096
NB = N // BN
NSC = 2
NTILE = 16
TOK_TILE = N // NTILE


def _scores_body(xb_ref, xd_ref, sc_ref, gi_ref, nm_s, ni_s):
    nb = pl.program_id(1)
    xb = xb_ref[0]
    xd = xd_ref[0]
    mb = xb / (jnp.sqrt(jnp.sum(xb * xb, axis=1, keepdims=True)) + 1e-6)
    md = xd / (jnp.sqrt(jnp.sum(xd * xd, axis=1, keepdims=True)) + 1e-6)
    s = lax.dot_general(md, mb, (((1,), (1,)), ((), ())))
    nmax = jnp.max(s, axis=0)
    nidx = jnp.argmax(s, axis=0)
    nm_s[pl.ds(nb * (BN // W), BN // W), :] = nmax.reshape(BN // W, W)
    ni_s[pl.ds(nb * (BN // W), BN // W), :] = nidx.astype(jnp.int32).reshape(
        BN // W, W)

    @pl.when(nb == NB - 1)
    def _():
        _select(nm_s[...], ni_s[...], sc_ref, gi_ref)


def _select(nm, ni, sc_ref, gi_ref):
    b = pl.program_id(0)
    ri = lax.broadcasted_iota(jnp.int32, (H, W), 0)
    ci = lax.broadcasted_iota(jnp.int32, (H, W), 1)
    is_dst = ((ri % SY) == 0) & ((ci % SX) == 0)
    bits = lax.bitcast_convert_type(nm, jnp.int32)
    v = bits ^ (lax.shift_right_arithmetic(bits, 31) & jnp.int32(0x7FFFFFFF))
    min32 = jnp.int32(-2147483648)
    v = jnp.where(is_dst, min32, v)
    r = jnp.int32(R_MERGE)
    cnt_pos = jnp.sum((v >= 0).astype(jnp.int32))
    bucket_pos = cnt_pos >= r
    in_bucket = ((v >= 0) == bucket_pos) & jnp.logical_not(is_dst)
    key = jnp.where(in_bucket, v & jnp.int32(0x7FFFFFFF), jnp.int32(-1))
    rr = jnp.where(bucket_pos, r, r - cnt_pos)

    def body(k, prefix):
        cand = prefix | lax.shift_left(jnp.int32(1), jnp.int32(30) - k)
        cnt = jnp.sum((key >= cand).astype(jnp.int32))
        return jnp.where(cnt >= rr, cand, prefix)

    t = lax.fori_loop(0, 31, body, jnp.int32(0))
    tv = jnp.where(bucket_pos, t, t | min32)
    not_dst = jnp.logical_not(is_dst)
    gt = (v > tv) & not_dst
    eq = (v == tv) & not_dst
    cnt_gt = jnp.sum(gt.astype(jnp.int32))
    need = (r - cnt_gt).astype(jnp.float32)
    e = eq.astype(jnp.float32)
    tri_incl = (ri <= ci).astype(jnp.float32)
    incl = lax.dot_general(e, tri_incl, (((1,), (0,)), ((), ())),
                           precision=lax.Precision.HIGHEST)
    row_tot = incl[:, W - 1:W]
    tri_strict = (ri > ci).astype(jnp.float32)
    off = lax.dot_general(tri_strict, row_tot, (((1,), (0,)), ((), ())),
                          precision=lax.Precision.HIGHEST)
    rank_excl = incl - e + off
    merged = gt | (eq & (rank_excl < need))
    dbin = (ri // SY) * (W // SX) + (ci // SX)
    sc_ref[0] = jnp.where(is_dst, dbin, jnp.where(merged, ni, jnp.int32(ND)))
    g = jnp.where(is_dst, dbin, jnp.where(merged, ni, jnp.int32(-1)))
    sel = g >= 0
    t = ri * W + ci
    gi_ref[0] = jnp.where(sel, b * NDP + g,
                          jnp.int32(NSC * NDP) + b * N + t)


def _scatter_body(xb_ref, si_ref, dm_ref, acc_ref, cnt_ref):
    nb = pl.program_id(1)

    @pl.when(nb == 0)
    def _():
        acc_ref[...] = jnp.zeros_like(acc_ref)
        cnt_ref[...] = jnp.zeros_like(cnt_ref)

    xb = xb_ref[0]
    si = jnp.broadcast_to(si_ref[0], (ND, BN))
    iota = lax.broadcasted_iota(jnp.int32, (ND, BN), 0)
    oht = (si == iota).astype(jnp.float32)
    acc_ref[pl.ds(0, ND), :] += lax.dot_general(
        oht, xb, (((1,), (0,)), ((), ())))
    cnt_ref[pl.ds(0, ND), :] += lax.dot_general(
        oht, jnp.ones((BN, 128), jnp.float32), (((1,), (0,)), ((), ())))

    @pl.when(nb == NB - 1)
    def _():
        dm_ref[0] = acc_ref[...] / jnp.maximum(cnt_ref[:, 0:1], 1.0)


def _sc_unmerge_body(gi_hbm, tab_hbm, out_hbm,
                     gb0, gb1, gixbuf, gs0, gs1, ws0, ws1):
    c = lax.axis_index("c")
    s = lax.axis_index("s")
    pltpu.sync_copy(gi_hbm.at[c, pl.ds(s * 8, 8)], gixbuf)

    base = s * TOK_TILE
    gb = (gb0, gb1)
    gsem = (gs0, gs1)
    wsem = (ws0, ws1)
    gld = [None, None]
    wr = [None, None]
    gld[0] = pltpu.async_copy(tab_hbm.at[gixbuf.at[0]], gb0, gs0)
    for k in range(8):
        cur = k % 2
        if k < 7:
            if wr[1 - cur] is not None:
                wr[1 - cur].wait()
            gld[1 - cur] = pltpu.async_copy(
                tab_hbm.at[gixbuf.at[k + 1]], gb[1 - cur], gsem[1 - cur])
        gld[cur].wait()
        wr[cur] = pltpu.async_copy(
            gb[cur], out_hbm.at[c, pl.ds(base + k * 128, 128)], wsem[cur])
    wr[0].wait()
    wr[1].wait()


_sc_unmerge = pl.kernel(
    _sc_unmerge_body,
    out_type=jax.ShapeDtypeStruct((NSC, N, C), jnp.float32),
    mesh=plsc.VectorSubcoreMesh(core_axis_name="c", subcore_axis_name="s"),
    scratch_types=[
        pltpu.VMEM((128, C), jnp.float32),
        pltpu.VMEM((128, C), jnp.float32),
        pltpu.VMEM((8, 128), jnp.int32),
        pltpu.SemaphoreType.DMA,
        pltpu.SemaphoreType.DMA,
        pltpu.SemaphoreType.DMA,
        pltpu.SemaphoreType.DMA,
    ],
)


def kernel(x):
    B = x.shape[0]
    x_dst = x.reshape(B, H // SY, SY, W // SX, SX, C)[:, :, 0, :, 0, :]
    x_dst = x_dst.reshape(B, ND, C)

    scat, gath = pl.pallas_call(
        _scores_body,
        grid=(B, NB),
        in_specs=[
            pl.BlockSpec((1, BN, C), lambda b, nb: (b, nb, 0)),
            pl.BlockSpec((1, ND, C), lambda b, nb: (b, 0, 0)),
        ],
        out_specs=[
            pl.BlockSpec((1, H, W), lambda b, nb: (b, 0, 0)),
            pl.BlockSpec((1, H, W), lambda b, nb: (b, 0, 0)),
        ],
        out_shape=[
            jax.ShapeDtypeStruct((B, H, W), jnp.int32),
            jax.ShapeDtypeStruct((B, H, W), jnp.int32),
        ],
        scratch_shapes=[
            pltpu.VMEM((H, W), jnp.float32),
            pltpu.VMEM((H, W), jnp.int32),
        ],
    )(x, x_dst)

    dm = pl.pallas_call(
        _scatter_body,
        grid=(B, NB),
        in_specs=[
            pl.BlockSpec((1, BN, C), lambda b, nb: (b, nb, 0)),
            pl.BlockSpec((1, 1, BN), lambda b, nb: (b * NB + nb, 0, 0)),
        ],
        out_specs=pl.BlockSpec((1, NDP, C), lambda b, nb: (b, 0, 0)),
        out_shape=jax.ShapeDtypeStruct((B, NDP, C), jnp.float32),
        scratch_shapes=[
            pltpu.VMEM((NDP, C), jnp.float32),
            pltpu.VMEM((NDP, 128), jnp.float32),
        ],
    )(x, scat.reshape(B * NB, 1, BN))

    tab = jnp.concatenate([dm.reshape(B * NDP, C), x.reshape(B * N, C)], 0)
    return _sc_unmerge(gath, tab)


# speedup vs baseline: 4.3819x; 1.0015x over previous
import jax
import jax.numpy as jnp
from jax import lax
from jax.experimental import pallas as pl
from jax.experimental.pallas import tpu as pltpu
from jax.experimental.pallas import tpu_sc as plsc

H = 128
W = 128
SY = 4
SX = 4
N = H * W
C = 256
ND = (H // SY) * (W // SX)
NDP = 1152
ZROW = 1024
OUT_N = 16384 + 8
R_MERGE = min(N // 2, N - ND)
BN = 4096
NB = N // BN
NSC = 2
NTILE = 16
TOK_TILE = N // NTILE


def _scores_body(xb_ref, xd_ref, sc_ref, gi_ref, nm_s, ni_s):
    nb = pl.program_id(1)
    xb = xb_ref[0]
    xd = xd_ref[0]
    mb = xb / (jnp.sqrt(jnp.sum(xb * xb, axis=1, keepdims=True)) + 1e-6)
    md = xd / (jnp.sqrt(jnp.sum(xd * xd, axis=1, keepdims=True)) + 1e-6)
    s = lax.dot_general(md, mb, (((1,), (1,)), ((), ())))
    nmax = jnp.max(s, axis=0)
    nidx = jnp.argmax(s, axis=0)
    nm_s[pl.ds(nb * (BN // W), BN // W), :] = nmax.reshape(BN // W, W)
    ni_s[pl.ds(nb * (BN // W), BN // W), :] = nidx.astype(jnp.int32).reshape(
        BN // W, W)

    @pl.when(nb == NB - 1)
    def _():
        _select(nm_s[...], ni_s[...], sc_ref, gi_ref)


def _select(nm, ni, sc_ref, gi_ref):
    b = pl.program_id(0)
    ri = lax.broadcasted_iota(jnp.int32, (H, W), 0)
    ci = lax.broadcasted_iota(jnp.int32, (H, W), 1)
    is_dst = ((ri % SY) == 0) & ((ci % SX) == 0)
    bits = lax.bitcast_convert_type(nm, jnp.int32)
    v = bits ^ (lax.shift_right_arithmetic(bits, 31) & jnp.int32(0x7FFFFFFF))
    min32 = jnp.int32(-2147483648)
    v = jnp.where(is_dst, min32, v)
    r = jnp.int32(R_MERGE)
    cnt_pos = jnp.sum((v >= 0).astype(jnp.int32))
    bucket_pos = cnt_pos >= r
    in_bucket = ((v >= 0) == bucket_pos) & jnp.logical_not(is_dst)
    key = jnp.where(in_bucket, v & jnp.int32(0x7FFFFFFF), jnp.int32(-1))
    rr = jnp.where(bucket_pos, r, r - cnt_pos)

    def body(k, prefix):
        cand = prefix | lax.shift_left(jnp.int32(1), jnp.int32(30) - k)
        cnt = jnp.sum((key >= cand).astype(jnp.int32))
        return jnp.where(cnt >= rr, cand, prefix)

    t = lax.fori_loop(0, 31, body, jnp.int32(0))
    tv = jnp.where(bucket_pos, t, t | min32)
    not_dst = jnp.logical_not(is_dst)
    gt = (v > tv) & not_dst
    eq = (v == tv) & not_dst
    cnt_gt = jnp.sum(gt.astype(jnp.int32))
    need = (r - cnt_gt).astype(jnp.float32)
    e = eq.astype(jnp.float32)
    tri_incl = (ri <= ci).astype(jnp.float32)
    incl = lax.dot_general(e, tri_incl, (((1,), (0,)), ((), ())),
                           precision=lax.Precision.HIGHEST)
    row_tot = incl[:, W - 1:W]
    tri_strict = (ri > ci).astype(jnp.float32)
    off = lax.dot_general(tri_strict, row_tot, (((1,), (0,)), ((), ())),
                          precision=lax.Precision.HIGHEST)
    rank_excl = incl - e + off
    merged = gt | (eq & (rank_excl < need))
    dbin = (ri // SY) * (W // SX) + (ci // SX)
    sc_ref[0] = jnp.where(is_dst, dbin, jnp.where(merged, ni, jnp.int32(ND)))
    g = jnp.where(is_dst, dbin, jnp.where(merged, ni, jnp.int32(-1)))
    sel = g >= 0
    t = ri * W + ci
    gi_ref[0] = jnp.where(sel, b * NDP + g,
                          jnp.int32(NSC * NDP) + b * N + t)


def _scatter_body(xb_ref, si_ref, dm_ref, acc_ref, cnt_ref):
    nb = pl.program_id(1)

    @pl.when(nb == 0)
    def _():
        acc_ref[...] = jnp.zeros_like(acc_ref)
        cnt_ref[...] = jnp.zeros_like(cnt_ref)

    xb = xb_ref[0]
    si = jnp.broadcast_to(si_ref[0], (ND, BN))
    iota = lax.broadcasted_iota(jnp.int32, (ND, BN), 0)
    oht = (si == iota).astype(jnp.float32)
    acc_ref[pl.ds(0, ND), :] += lax.dot_general(
        oht, xb, (((1,), (0,)), ((), ())))
    cnt_ref[pl.ds(0, ND), :] += lax.dot_general(
        oht.astype(jnp.bfloat16), jnp.ones((BN, 128), jnp.bfloat16),
        (((1,), (0,)), ((), ())), preferred_element_type=jnp.float32)

    @pl.when(nb == NB - 1)
    def _():
        dm_ref[0] = acc_ref[...] / jnp.maximum(cnt_ref[:, 0:1], 1.0)


def _sc_unmerge_body(gi_hbm, tab_hbm, out_hbm,
                     gb0, gb1, gixbuf, gs0, gs1, ws0, ws1):
    c = lax.axis_index("c")
    s = lax.axis_index("s")
    pltpu.sync_copy(gi_hbm.at[c, pl.ds(s * 8, 8)], gixbuf)

    base = s * TOK_TILE
    gb = (gb0, gb1)
    gsem = (gs0, gs1)
    wsem = (ws0, ws1)
    gld = [None, None]
    wr = [None, None]
    gld[0] = pltpu.async_copy(tab_hbm.at[gixbuf.at[0]], gb0, gs0)
    for k in range(8):
        cur = k % 2
        if k < 7:
            if wr[1 - cur] is not None:
                wr[1 - cur].wait()
            gld[1 - cur] = pltpu.async_copy(
                tab_hbm.at[gixbuf.at[k + 1]], gb[1 - cur], gsem[1 - cur])
        gld[cur].wait()
        wr[cur] = pltpu.async_copy(
            gb[cur], out_hbm.at[c, pl.ds(base + k * 128, 128)], wsem[cur])
    wr[0].wait()
    wr[1].wait()


_sc_unmerge = pl.kernel(
    _sc_unmerge_body,
    out_type=jax.ShapeDtypeStruct((NSC, N, C), jnp.float32),
    mesh=plsc.VectorSubcoreMesh(core_axis_name="c", subcore_axis_name="s"),
    scratch_types=[
        pltpu.VMEM((128, C), jnp.float32),
        pltpu.VMEM((128, C), jnp.float32),
        pltpu.VMEM((8, 128), jnp.int32),
        pltpu.SemaphoreType.DMA,
        pltpu.SemaphoreType.DMA,
        pltpu.SemaphoreType.DMA,
        pltpu.SemaphoreType.DMA,
    ],
)


def kernel(x):
    B = x.shape[0]
    x_dst = x.reshape(B, H // SY, SY, W // SX, SX, C)[:, :, 0, :, 0, :]
    x_dst = x_dst.reshape(B, ND, C)

    scat, gath = pl.pallas_call(
        _scores_body,
        grid=(B, NB),
        in_specs=[
            pl.BlockSpec((1, BN, C), lambda b, nb: (b, nb, 0)),
            pl.BlockSpec((1, ND, C), lambda b, nb: (b, 0, 0)),
        ],
        out_specs=[
            pl.BlockSpec((1, H, W), lambda b, nb: (b, 0, 0)),
            pl.BlockSpec((1, H, W), lambda b, nb: (b, 0, 0)),
        ],
        out_shape=[
            jax.ShapeDtypeStruct((B, H, W), jnp.int32),
            jax.ShapeDtypeStruct((B, H, W), jnp.int32),
        ],
        scratch_shapes=[
            pltpu.VMEM((H, W), jnp.float32),
            pltpu.VMEM((H, W), jnp.int32),
        ],
    )(x, x_dst)

    dm = pl.pallas_call(
        _scatter_body,
        grid=(B, NB),
        in_specs=[
            pl.BlockSpec((1, BN, C), lambda b, nb: (b, nb, 0)),
            pl.BlockSpec((1, 1, BN), lambda b, nb: (b * NB + nb, 0, 0)),
        ],
        out_specs=pl.BlockSpec((1, NDP, C), lambda b, nb: (b, 0, 0)),
        out_shape=jax.ShapeDtypeStruct((B, NDP, C), jnp.float32),
        scratch_shapes=[
            pltpu.VMEM((NDP, C), jnp.float32),
            pltpu.VMEM((NDP, 128), jnp.float32),
        ],
    )(x, scat.reshape(B * NB, 1, BN))

    tab = jnp.concatenate([dm.reshape(B * NDP, C), x.reshape(B * N, C)], 0)
    return _sc_unmerge(gath, tab)
